# baseline (speedup 1.0000x reference)
import jax
import jax.numpy as jnp
from jax.experimental import pallas as pl

_BB = 512
_CPAD = 1024


def _block_kernel(lab_ref, f_ref, c_ref, out_ref):
    i = pl.program_id(0)
    lab = lab_ref[0].astype(jnp.int16)
    col = jax.lax.broadcasted_iota(jnp.int16, (_BB, _CPAD), 1)
    onehot = (col == lab).astype(jnp.bfloat16)
    nsp = 8
    ch = f_ref.shape[1] // nsp
    part = jnp.zeros((1, 1), jnp.float32)
    for n in range(nsp):
        bc = jnp.dot(onehot, c_ref[:, n * ch:(n + 1) * ch],
                     preferred_element_type=jnp.float32)
        d = f_ref[:, n * ch:(n + 1) * ch] - bc
        part = part + jnp.sum(d * d, keepdims=True)

    @pl.when(i == 0)
    def _init():
        out_ref[...] = jnp.zeros((1, 1), jnp.float32)

    out_ref[...] += part


def kernel(features, labels, centers):
    batch, feat = features.shape
    nclass = centers.shape[0]
    g = batch // _BB
    lab3 = labels.astype(jnp.int32).reshape(g, _BB, 1)
    cpad = jnp.pad(centers, ((0, _CPAD - nclass), (0, 0))).astype(jnp.bfloat16)
    total = pl.pallas_call(
        _block_kernel,
        grid=(g,),
        in_specs=[
            pl.BlockSpec((1, _BB, 1), lambda i: (i, 0, 0)),
            pl.BlockSpec((_BB, feat), lambda i: (i, 0)),
            pl.BlockSpec((_CPAD, feat), lambda i: (0, 0)),
        ],
        out_specs=pl.BlockSpec((1, 1), lambda i: (0, 0)),
        out_shape=jax.ShapeDtypeStruct((1, 1), jnp.float32),
    )(lab3, features, cpad)
    return (0.5 / batch) * total[0, 0]


# speedup vs baseline: 1.0914x; 1.0914x over previous
import jax
import jax.numpy as jnp
from jax.experimental import pallas as pl
from jax.experimental.pallas import tpu as pltpu

_BB = 1024
_CPAD = 1024


def _block_kernel(lab_ref, f_ref, c_ref, out_ref, cbf_ref):
    i = pl.program_id(0)
    nclass = c_ref.shape[0]

    @pl.when(i == 0)
    def _stage_centers():
        cbf_ref[:nclass, :] = c_ref[...].astype(jnp.bfloat16)
        cbf_ref[nclass:, :] = jnp.zeros(
            (_CPAD - nclass, c_ref.shape[1]), jnp.bfloat16)

    lab = lab_ref[0].astype(jnp.int16)
    col = jax.lax.broadcasted_iota(jnp.int16, (_BB, _CPAD), 1)
    onehot = (col == lab).astype(jnp.bfloat16)
    nsp = 8
    ch = f_ref.shape[1] // nsp
    part = jnp.zeros((1, 1), jnp.float32)
    for n in range(nsp):
        bc = jnp.dot(onehot, cbf_ref[:, n * ch:(n + 1) * ch],
                     preferred_element_type=jnp.float32)
        d = f_ref[:, n * ch:(n + 1) * ch] - bc
        part = part + jnp.sum(d * d, keepdims=True)

    @pl.when(i == 0)
    def _init():
        out_ref[...] = jnp.zeros((1, 1), jnp.float32)

    out_ref[...] += part


def kernel(features, labels, centers):
    batch, feat = features.shape
    nclass = centers.shape[0]
    g = batch // _BB
    lab3 = labels.astype(jnp.int32).reshape(g, _BB, 1)
    total = pl.pallas_call(
        _block_kernel,
        grid=(g,),
        in_specs=[
            pl.BlockSpec((1, _BB, 1), lambda i: (i, 0, 0)),
            pl.BlockSpec((_BB, feat), lambda i: (i, 0)),
            pl.BlockSpec((nclass, feat), lambda i: (0, 0)),
        ],
        out_specs=pl.BlockSpec((1, 1), lambda i: (0, 0)),
        out_shape=jax.ShapeDtypeStruct((1, 1), jnp.float32),
        scratch_shapes=[pltpu.VMEM((_CPAD, feat), jnp.bfloat16)],
    )(lab3, features, centers)
    return (0.5 / batch) * total[0, 0]
